# Initial kernel scaffold; baseline (speedup 1.0000x reference)
import jax
import jax.numpy as jnp
from jax.experimental import pallas as pl


def kernel(x, edge_index, W_self, W_neigh, b):
    raise NotImplementedError("write your pallas kernel here")


# speedup vs baseline: 5.1702x; 5.1702x over previous
import functools

import jax
import jax.numpy as jnp
from jax import lax
from jax.experimental import pallas as pl
from jax.experimental.pallas import tpu as pltpu
from jax.experimental.pallas import tpu_sc as plsc

NC = 2
NS = 16
NW = NC * NS
CH = 128
LANES = 16


def _sc_aggregate(x, src_t, dst_t, n_pad, d, chunks_per_tile):
    zr = n_pad // NS
    zb = ((zr + LANES - 1) // LANES) * LANES

    def body(x_hbm, src_hbm, dst_hbm, agg_hbm, deg0_hbm, deg1_hbm,
             src_v, dst_v, buf0, buf1, ones_v, zeros_v,
             acc_sh, deg_sh, sem0, sem1):
        c = lax.axis_index("c")
        s = lax.axis_index("s")
        wid = c * NS + s

        zero16 = jnp.zeros((LANES,), jnp.float32)
        one16 = jnp.ones((LANES,), jnp.float32)

        def fill_row(i, _):
            for k in range(d // LANES):
                buf0[i, pl.ds(k * LANES, LANES)] = zero16
            return 0

        lax.fori_loop(0, CH, fill_row, 0)
        for k in range(CH // LANES):
            ones_v[pl.ds(k * LANES, LANES)] = one16
        for k in range(zb // LANES):
            zeros_v[pl.ds(k * LANES, LANES)] = zero16

        row0 = s * zr
        off = 0
        while off < zr:
            step = min(CH, zr - off)
            pltpu.sync_copy(buf0.at[pl.ds(0, step), :],
                            acc_sh.at[pl.ds(row0 + off, step), :])
            off += step
        pltpu.sync_copy(zeros_v.at[pl.ds(0, zr)], deg_sh.at[pl.ds(row0, zr)])
        plsc.subcore_barrier()

        hcpt = chunks_per_tile // 2
        for h in range(2):
            pltpu.sync_copy(src_hbm.at[wid, pl.ds(h * hcpt, hcpt), :], src_v)
            pltpu.sync_copy(dst_hbm.at[wid, pl.ds(h * hcpt, hcpt), :], dst_v)

            pltpu.async_copy(x_hbm.at[src_v.at[0]], buf0, sem0)
            pltpu.async_copy(x_hbm.at[src_v.at[1]], buf1, sem1)

            def step(i, _):
                j0 = 2 * i
                pltpu.make_async_copy(
                    x_hbm.at[src_v.at[j0]], buf0, sem0).wait()
                pltpu.sync_copy(buf0, acc_sh.at[dst_v.at[j0]], add=True)
                pltpu.sync_copy(ones_v, deg_sh.at[dst_v.at[j0]], add=True)
                pltpu.async_copy(x_hbm.at[src_v.at[j0 + 2]], buf0, sem0)
                pltpu.make_async_copy(
                    x_hbm.at[src_v.at[j0 + 1]], buf1, sem1).wait()
                pltpu.sync_copy(buf1, acc_sh.at[dst_v.at[j0 + 1]], add=True)
                pltpu.sync_copy(ones_v, deg_sh.at[dst_v.at[j0 + 1]], add=True)
                pltpu.async_copy(x_hbm.at[src_v.at[j0 + 3]], buf1, sem1)
                return 0

            lax.fori_loop(0, hcpt // 2 - 1, step, 0)
            last = hcpt - 2
            pltpu.make_async_copy(x_hbm.at[src_v.at[last]], buf0, sem0).wait()
            pltpu.sync_copy(buf0, acc_sh.at[dst_v.at[last]], add=True)
            pltpu.sync_copy(ones_v, deg_sh.at[dst_v.at[last]], add=True)
            pltpu.make_async_copy(
                x_hbm.at[src_v.at[last + 1]], buf1, sem1).wait()
            pltpu.sync_copy(buf1, acc_sh.at[dst_v.at[last + 1]], add=True)
            pltpu.sync_copy(ones_v, deg_sh.at[dst_v.at[last + 1]], add=True)

        plsc.subcore_barrier()

        pltpu.sync_copy(acc_sh.at[pl.ds(row0, zr), :],
                        agg_hbm.at[c, pl.ds(row0, zr), :])

        pltpu.sync_copy(deg_sh.at[pl.ds(row0, zr)], zeros_v.at[pl.ds(0, zr)])

        @pl.when(c == 0)
        def _():
            pltpu.sync_copy(zeros_v.at[pl.ds(0, zr)],
                            deg0_hbm.at[pl.ds(row0, zr)])

        @pl.when(c == 1)
        def _():
            pltpu.sync_copy(zeros_v.at[pl.ds(0, zr)],
                            deg1_hbm.at[pl.ds(row0, zr)])

    mesh = plsc.VectorSubcoreMesh(core_axis_name="c", subcore_axis_name="s")
    fn = pl.kernel(
        body,
        out_type=[
            jax.ShapeDtypeStruct((NC, n_pad, d), jnp.float32),
            jax.ShapeDtypeStruct((n_pad,), jnp.float32),
            jax.ShapeDtypeStruct((n_pad,), jnp.float32),
        ],
        mesh=mesh,
        scratch_types=[
            pltpu.VMEM((chunks_per_tile // 2, CH), jnp.int32),
            pltpu.VMEM((chunks_per_tile // 2, CH), jnp.int32),
            pltpu.VMEM((CH, d), jnp.float32),
            pltpu.VMEM((CH, d), jnp.float32),
            pltpu.VMEM((CH,), jnp.float32),
            pltpu.VMEM((zb,), jnp.float32),
            pltpu.VMEM_SHARED((n_pad, d), jnp.float32),
            pltpu.VMEM_SHARED((n_pad,), jnp.float32),
            pltpu.SemaphoreType.DMA,
            pltpu.SemaphoreType.DMA,
        ],
    )
    return fn(x, src_t, dst_t)


def _tc_combine(x_pad, agg_p, deg0, deg1, w_self, w_neigh, b2, n_pad, d):
    nblk = 8
    r = n_pad // nblk

    def body(x_ref, p_ref, d0_ref, d1_ref, ws_ref, wn_ref, b_ref, o_ref):
        p = p_ref[0] + p_ref[1]
        deg = d0_ref[...] + d1_ref[...]
        agg = p / jnp.maximum(deg, 1.0)
        o_ref[...] = (
            jnp.dot(x_ref[...], ws_ref[...], preferred_element_type=jnp.float32)
            + jnp.dot(agg, wn_ref[...], preferred_element_type=jnp.float32)
            + b_ref[...]
        )

    return pl.pallas_call(
        body,
        grid=(nblk,),
        in_specs=[
            pl.BlockSpec((r, d), lambda i: (i, 0)),
            pl.BlockSpec((NC, r, d), lambda i: (0, i, 0)),
            pl.BlockSpec((r, 1), lambda i: (i, 0)),
            pl.BlockSpec((r, 1), lambda i: (i, 0)),
            pl.BlockSpec((d, d), lambda i: (0, 0)),
            pl.BlockSpec((d, d), lambda i: (0, 0)),
            pl.BlockSpec((1, d), lambda i: (0, 0)),
        ],
        out_specs=pl.BlockSpec((r, d), lambda i: (i, 0)),
        out_shape=jax.ShapeDtypeStruct((n_pad, d), jnp.float32),
    )(x_pad, agg_p, deg0, deg1, w_self, w_neigh, b2)


def kernel(x, edge_index, W_self, W_neigh, b):
    n, d = x.shape
    e = edge_index.shape[1]

    n_pad = ((n + 1 + NS * 8 - 1) // (NS * 8)) * (NS * 8)
    chunks_per_tile = -(-e // (NW * CH))
    chunks_per_tile = ((chunks_per_tile + 3) // 4) * 4
    e_pad = NW * chunks_per_tile * CH

    src = edge_index[0]
    dst = edge_index[1]
    pad = e_pad - e
    src_t = jnp.concatenate(
        [src, jnp.zeros((pad,), jnp.int32)]).reshape(NW, chunks_per_tile, CH)
    dst_t = jnp.concatenate(
        [dst, jnp.full((pad,), n, jnp.int32)]).reshape(NW, chunks_per_tile, CH)

    agg_p, deg0, deg1 = _sc_aggregate(x, src_t, dst_t, n_pad, d,
                                      chunks_per_tile)

    x_pad = jnp.pad(x, ((0, n_pad - n), (0, 0)))
    out = _tc_combine(x_pad, agg_p,
                      deg0.reshape(n_pad, 1), deg1.reshape(n_pad, 1),
                      W_self, W_neigh, b.reshape(1, d), n_pad, d)
    return out[:n]
